# Initial kernel scaffold; baseline (speedup 1.0000x reference)
import jax
import jax.numpy as jnp
from jax.experimental import pallas as pl


def kernel(p, x, o, Wq, bq, Wk, bk, Wv, bv, Wp1, bp1, gln_p, bln_p, Wp2, bp2, g1, be1, Ww1, bw1, g2, be2, Ww2, bw2):
    raise NotImplementedError("write your pallas kernel here")


# speedup vs baseline: 4.8315x; 4.8315x over previous
import jax
import jax.numpy as jnp
from jax import lax
from jax.experimental import pallas as pl
from jax.experimental.pallas import tpu as pltpu
from jax.experimental.pallas import tpu_sc as plsc

N = 4096
C = 128
H = 8
NS = 16
MID = C // H
WSH = C // 8
KVP = 3 * C

BQ1 = 256
NB1 = N // BQ1
BQ2 = 128
NB2 = N // BQ2

NW = 32
QW = N // NW
NCHUNK = N // 16

_HP = lax.Precision.HIGHEST


def _proj_body(pp_ref, ppT_ref, x_ref, Wq_ref, bq_ref, Wk_ref, bk_ref,
               Wv_ref, bv_ref, xq_ref, kvp_ref, d2_ref, t_ref):
    xx = x_ref[...]
    xq_ref[...] = jnp.dot(xx, Wq_ref[...]) + bq_ref[...]
    xk = jnp.dot(xx, Wk_ref[...]) + bk_ref[...]
    xv = jnp.dot(xx, Wv_ref[...]) + bv_ref[...]

    pb = pp_ref[...]
    pT = ppT_ref[...]
    kvp_ref[...] = jnp.concatenate(
        [xk, xv, pb, jnp.zeros((BQ1, C - 8), jnp.float32)], axis=1)

    pn_b = jnp.sum(pb * pb, axis=1, keepdims=True)
    pn_all = jnp.sum(pT * pT, axis=0, keepdims=True)
    d2 = pn_b + pn_all - 2.0 * jnp.dot(pb, pT)
    d2_ref[...] = d2
    m16 = jnp.min(d2.reshape(BQ1, 16, N // 16), axis=2)
    t = jnp.max(m16, axis=1, keepdims=True)
    t_ref[...] = jnp.broadcast_to(t, (BQ1, 16))


def _sc_body(d2_h, kvp_h, kvpg_h, row, idxall, gidx, bkvp, sem0):
    wid = lax.axis_index("s") * 2 + lax.axis_index("c")
    base_q = wid * QW

    lanes = lax.iota(jnp.int32, 16)
    inf16 = jnp.full((16,), jnp.inf, jnp.float32)
    zero16 = jnp.zeros((16,), jnp.int32)

    def row_body(r, carry):
        pltpu.sync_copy(d2_h.at[base_q + r], row)

        def chunk_body(j, best):
            bk_, bi_ = best
            v = row[pl.ds(j * 16, 16)]
            ck, ci = plsc.sort_key_val(v, j * 16 + lanes)
            rk = lax.rev(ck, (0,))
            ri = lax.rev(ci, (0,))
            mm = bk_ <= rk
            sk, si = plsc.sort_key_val(jnp.where(mm, bk_, rk),
                                       jnp.where(mm, bi_, ri))
            return (sk, si)

        _, ii = lax.fori_loop(0, NCHUNK, chunk_body, (inf16, zero16))
        idxall[pl.ds(r * 16, 16)] = ii
        return carry

    lax.fori_loop(0, QW, row_body, jnp.int32(0))

    for cix in range(16):
        for k8 in range(8):
            gidx[pl.ds(k8 * 16, 16)] = idxall[pl.ds(cix * 128 + k8 * 16, 16)]
        pltpu.async_copy(kvp_h.at[gidx], bkvp, sem0).wait()
        ob = base_q * NS + cix * 128
        pltpu.sync_copy(bkvp, kvpg_h.at[pl.ds(ob, 128)])


def _attn_body(xq_ref, p16_ref, kvpg_ref,
               Wp1_ref, bp1_ref, glnp_ref, blnp_ref, Wp2_ref, bp2_ref,
               G_ref, g1_ref, be1_ref, BW1_ref, bw1_ref,
               g2_ref, be2_ref, BW2_ref, bw2_ref, out_ref):
    R = BQ2 * NS
    xq = xq_ref[...]
    pq = p16_ref[...]
    g = kvpg_ref[...]
    xkg = g[:, :C]
    xvg = g[:, C:2 * C]
    pg = g[:, 2 * C:2 * C + 16]

    pq_rep = jnp.broadcast_to(pq[:, None, :], (BQ2, NS, 16)).reshape(R, 16)
    p_r = pg - pq_rep

    t = jnp.dot(p_r, Wp1_ref[...]) + bp1_ref[...]
    m = jnp.sum(t, axis=1, keepdims=True) * (1.0 / 3.0)
    mask3 = (lax.broadcasted_iota(jnp.int32, (1, 16), 1) < 3).astype(t.dtype)
    d = (t - m) * mask3
    v = jnp.sum(d * d, axis=1, keepdims=True) * (1.0 / 3.0)
    tn = d * lax.rsqrt(v + 1e-5) * glnp_ref[...] + blnp_ref[...]
    tn = jnp.maximum(tn, 0.0)
    prf = jnp.dot(tn, Wp2_ref[...]) + bp2_ref[...]

    xq_rep = jnp.broadcast_to(xq[:, None, :], (BQ2, NS, C)).reshape(R, C)
    r_qk = xkg + prf - xq_rep

    G = G_ref[...]
    s1 = jnp.dot(r_qk, G, precision=_HP) * (1.0 / MID)
    d1 = r_qk - s1
    v1 = jnp.dot(d1 * d1, G, precision=_HP) * (1.0 / MID)
    y = d1 * lax.rsqrt(v1 + 1e-5) * g1_ref[...] + be1_ref[...]
    y = jnp.maximum(y, 0.0)
    y = jnp.dot(y, BW1_ref[...]) + bw1_ref[...]

    s2 = jnp.dot(y, G, precision=_HP) * (1.0 / WSH)
    d2_ = y - s2
    v2 = jnp.dot(d2_ * d2_, G, precision=_HP) * (1.0 / WSH)
    y2 = d2_ * lax.rsqrt(v2 + 1e-5) * g2_ref[...] + be2_ref[...]
    y2 = jnp.maximum(y2, 0.0)
    wf = jnp.dot(y2, BW2_ref[...]) + bw2_ref[...]

    wm = jnp.dot(wf, G, precision=_HP) * (1.0 / WSH)
    wm3 = wm.reshape(BQ2, NS, C)
    mx = jnp.max(wm3, axis=1, keepdims=True)
    e = jnp.exp(wm3 - mx)
    ssum = jnp.sum(e, axis=1, keepdims=True)
    w3 = e / ssum

    contrib = (xvg + prf).reshape(BQ2, NS, C) * w3
    out_ref[...] = jnp.sum(contrib, axis=1)


def kernel(p, x, o, Wq, bq, Wk, bk, Wv, bv, Wp1, bp1, gln_p, bln_p, Wp2, bp2,
           g1, be1, Ww1, bw1, g2, be2, Ww2, bw2):
    del o
    f32 = jnp.float32
    pp = jnp.zeros((N, 8), f32).at[:, :3].set(p)
    ppT = pp.T
    p16 = jnp.zeros((N, 16), f32).at[:, :3].set(p)

    full = lambda shp: pl.BlockSpec(shp, lambda i: (0,) * len(shp))

    xq, kvp, d2, thr = pl.pallas_call(
        _proj_body,
        grid=(NB1,),
        in_specs=[
            pl.BlockSpec((BQ1, 8), lambda i: (i, 0)),
            full((8, N)),
            pl.BlockSpec((BQ1, C), lambda i: (i, 0)),
            full((C, C)), full((1, C)),
            full((C, C)), full((1, C)),
            full((C, C)), full((1, C)),
        ],
        out_specs=[
            pl.BlockSpec((BQ1, C), lambda i: (i, 0)),
            pl.BlockSpec((BQ1, KVP), lambda i: (i, 0)),
            pl.BlockSpec((BQ1, N), lambda i: (i, 0)),
            pl.BlockSpec((BQ1, 16), lambda i: (i, 0)),
        ],
        out_shape=[
            jax.ShapeDtypeStruct((N, C), f32),
            jax.ShapeDtypeStruct((N, KVP), f32),
            jax.ShapeDtypeStruct((N, N), f32),
            jax.ShapeDtypeStruct((N, 16), f32),
        ],
    )(pp, ppT, x, Wq, bq.reshape(1, C), Wk, bk.reshape(1, C),
      Wv, bv.reshape(1, C))
    del thr

    mesh = plsc.VectorSubcoreMesh(core_axis_name="c", subcore_axis_name="s")
    sc_fn = pl.kernel(
        _sc_body,
        out_type=jax.ShapeDtypeStruct((N * NS, KVP), f32),
        mesh=mesh,
        compiler_params=pltpu.CompilerParams(needs_layout_passes=False),
        scratch_types=[
            pltpu.VMEM((N,), f32),
            pltpu.VMEM((QW * NS,), jnp.int32),
            pltpu.VMEM((128,), jnp.int32),
            pltpu.VMEM((128, KVP), f32),
            pltpu.SemaphoreType.DMA,
        ],
    )
    kvpg = sc_fn(d2, kvp)

    Wp1p = jnp.zeros((16, 16), f32).at[:3, :3].set(Wp1)
    bp1p = jnp.zeros((1, 16), f32).at[0, :3].set(bp1)
    glnp = jnp.zeros((1, 16), f32).at[0, :3].set(gln_p)
    blnp = jnp.zeros((1, 16), f32).at[0, :3].set(bln_p)
    Wp2p = jnp.zeros((16, C), f32).at[:3, :].set(Wp2)
    gidx = jnp.arange(C) // MID
    G = (gidx[:, None] == gidx[None, :]).astype(f32)
    BW1 = jnp.kron(jnp.eye(H, dtype=f32), Ww1)
    BW2 = jnp.kron(jnp.eye(H, dtype=f32), Ww2)

    out = pl.pallas_call(
        _attn_body,
        grid=(NB2,),
        in_specs=[
            pl.BlockSpec((BQ2, C), lambda i: (i, 0)),
            pl.BlockSpec((BQ2, 16), lambda i: (i, 0)),
            pl.BlockSpec((BQ2 * NS, KVP), lambda i: (i, 0)),
            full((16, 16)), full((1, 16)), full((1, 16)), full((1, 16)),
            full((16, C)), full((1, C)),
            full((C, C)), full((1, C)), full((1, C)),
            full((C, C)), full((1, C)),
            full((1, C)), full((1, C)),
            full((C, C)), full((1, C)),
        ],
        out_specs=pl.BlockSpec((BQ2, C), lambda i: (i, 0)),
        out_shape=jax.ShapeDtypeStruct((N, C), f32),
    )(xq, p16, kvpg,
      Wp1p, bp1p, glnp, blnp, Wp2p, bp2.reshape(1, C),
      G, jnp.tile(g1, H).reshape(1, C), jnp.tile(be1, H).reshape(1, C),
      BW1, jnp.tile(bw1, H).reshape(1, C),
      jnp.tile(g2, H).reshape(1, C), jnp.tile(be2, H).reshape(1, C),
      BW2, jnp.tile(bw2, H).reshape(1, C))
    return out
